# Initial kernel scaffold; baseline (speedup 1.0000x reference)
import jax
import jax.numpy as jnp
from jax.experimental import pallas as pl


def kernel(h, m, wh, wm, bh, bm, norm, edge_index):
    raise NotImplementedError("write your pallas kernel here")


# speedup vs baseline: 6.4478x; 6.4478x over previous
import functools

import jax
import jax.numpy as jnp
from jax import lax
from jax.experimental import pallas as pl
from jax.experimental.pallas import tpu as pltpu
from jax.experimental.pallas import tpu_sc as plsc

N = 10000
E = 320000
D = 128

_NC = 2
_NS = 16
_NW = _NC * _NS
_B = 128
_NBATCH = E // _B
_ITERS = -(-_NBATCH // _NW)
_RHI = 640
_RLO = N - (_NS - 1) * _RHI


def _mm_body(h_ref, m_ref, wh_ref, wm_ref, norm_ref, h2_ref, m2_ref):
    h2 = jnp.dot(h_ref[...], wh_ref[...], preferred_element_type=jnp.float32)
    h2_ref[...] = h2 * norm_ref[...]
    m2_ref[...] = jnp.dot(m_ref[...], wm_ref[...], preferred_element_type=jnp.float32)


_mm = pl.pallas_call(
    _mm_body,
    out_shape=(
        jax.ShapeDtypeStruct((N, D), jnp.float32),
        jax.ShapeDtypeStruct((N, D), jnp.float32),
    ),
)


def _final_body(agg_ref, m2_ref, norm_ref, bh_ref, bm_ref, out_ref):
    s = (agg_ref[0] + agg_ref[1]) * norm_ref[...]
    s = s + bh_ref[...] + m2_ref[...] + bm_ref[...]
    out_ref[...] = jnp.maximum(s, 0.0)


_final = pl.pallas_call(
    _final_body,
    out_shape=jax.ShapeDtypeStruct((N, D), jnp.float32),
)


_mesh = plsc.VectorSubcoreMesh(core_axis_name="c", subcore_axis_name="s")


@functools.partial(
    pl.kernel,
    out_type=jax.ShapeDtypeStruct((_NC, N, D), jnp.float32),
    mesh=_mesh,
    scratch_types=[
        pltpu.VMEM((_B,), jnp.int32),
        pltpu.VMEM((_B,), jnp.int32),
        pltpu.VMEM((_B, D), jnp.float32),
        pltpu.VMEM_SHARED((N, D), jnp.float32),
        pltpu.SemaphoreType.DMA,
    ],
)
def _sc_agg(h2_hbm, src_hbm, dst_hbm, zeros_hbm, out_hbm,
            src_v, dst_v, rows_v, acc_sh, sem):
    cid = lax.axis_index("c")
    sid = lax.axis_index("s")
    w = cid * _NS + sid

    @pl.when(sid < _NS - 1)
    def _():
        pltpu.sync_copy(zeros_hbm, acc_sh.at[pl.ds(sid * _RHI, _RHI)])

    @pl.when(sid == _NS - 1)
    def _():
        pltpu.sync_copy(zeros_hbm.at[pl.ds(0, _RLO)],
                        acc_sh.at[pl.ds(sid * _RHI, _RLO)])

    plsc.subcore_barrier()

    def body(i, carry):
        bidx = w + i * _NW

        @pl.when(bidx < _NBATCH)
        def _():
            off = bidx * _B
            pltpu.sync_copy(src_hbm.at[pl.ds(off, _B)], src_v)
            pltpu.sync_copy(dst_hbm.at[pl.ds(off, _B)], dst_v)
            pltpu.async_copy(h2_hbm.at[src_v], rows_v, sem).wait()
            pltpu.sync_copy(rows_v, acc_sh.at[dst_v], add=True)

        return carry

    lax.fori_loop(0, _ITERS, body, 0)
    plsc.subcore_barrier()

    @pl.when(sid < _NS - 1)
    def _():
        pltpu.sync_copy(acc_sh.at[pl.ds(sid * _RHI, _RHI)],
                        out_hbm.at[cid, pl.ds(sid * _RHI, _RHI)])

    @pl.when(sid == _NS - 1)
    def _():
        pltpu.sync_copy(acc_sh.at[pl.ds(sid * _RHI, _RLO)],
                        out_hbm.at[cid, pl.ds(sid * _RHI, _RLO)])


def kernel(h, m, wh, wm, bh, bm, norm, edge_index):
    h2, m2 = _mm(h, m, wh, wm, norm)
    src = edge_index[0]
    dst = edge_index[1]
    zeros = jnp.zeros((_RHI, D), dtype=jnp.float32)
    agg = _sc_agg(h2, src, dst, zeros)
    return _final(agg, m2, norm, bh.reshape(1, D), bm.reshape(1, D))
